# Initial kernel scaffold; baseline (speedup 1.0000x reference)
import jax
import jax.numpy as jnp
from jax.experimental import pallas as pl


def kernel(x_s, x_t, edge_index, edge_attr, x_u, W1, b1, W2, b2, W3, b3, W4, b4, g):
    raise NotImplementedError("write your pallas kernel here")


# speedup vs baseline: 1.2016x; 1.2016x over previous
import functools

import jax
import jax.numpy as jnp
from jax.experimental import pallas as pl
from jax.experimental.pallas import tpu as pltpu

LEAKY_SLOPE = 0.2
EPS_F32 = 1.1920929e-07

_INTERP = False


def _leaky(x):
    return jnp.where(x >= 0, x, LEAKY_SLOPE * x)


def _edge_body(xtg_ref, ea_ref, w1_ref, b1_ref, w2_ref, b2_ref, msg_ref):
    a = (
        jnp.dot(xtg_ref[...], w1_ref[0:128, :], preferred_element_type=jnp.float32)
        + jnp.dot(ea_ref[...], w1_ref[128:144, :], preferred_element_type=jnp.float32)
        + b1_ref[...]
    )
    msg_ref[...] = (
        jnp.dot(_leaky(a), w2_ref[...], preferred_element_type=jnp.float32)
        + b2_ref[...]
    )


def _edge_stage(xtg, ea, W1, b1, W2, b2, tile=1280):
    E = xtg.shape[0]
    grid = E // tile
    return pl.pallas_call(
        _edge_body,
        grid=(grid,),
        in_specs=[
            pl.BlockSpec((tile, 128), lambda i: (i, 0)),
            pl.BlockSpec((tile, 16), lambda i: (i, 0)),
            pl.BlockSpec((144, 144), lambda i: (0, 0)),
            pl.BlockSpec((1, 144), lambda i: (0, 0)),
            pl.BlockSpec((144, 144), lambda i: (0, 0)),
            pl.BlockSpec((1, 144), lambda i: (0, 0)),
        ],
        out_specs=pl.BlockSpec((tile, 144), lambda i: (i, 0)),
        out_shape=jax.ShapeDtypeStruct((E, 144), jnp.float32),
        interpret=_INTERP,
    )(xtg, ea, W1, b1.reshape(1, -1), W2, b2.reshape(1, -1))


def _node_body(s1_ref, s2_ref, s3_ref, s4_ref, cnt_ref, xs_ref, xu_ref,
               w3_ref, b3_ref, w4_ref, b4_ref, g_ref, out_ref):
    cnt = cnt_ref[...]
    inv = 1.0 / cnt
    mean = s1_ref[...] * inv
    m2 = s2_ref[...] * inv
    m3 = s3_ref[...] * inv
    m4 = s4_ref[...] * inv
    mean2 = mean * mean
    var = _leaky(m2 - mean2)
    std = jnp.sqrt(var + 1e-6)
    c3 = m3 - 3.0 * mean * m2 + 2.0 * mean * mean2
    c4 = m4 - 4.0 * mean * m3 + 6.0 * mean2 * m2 - 3.0 * mean2 * mean2
    inv_std = 1.0 / std
    inv_std2 = inv_std * inv_std
    skew = c3 * inv_std2 * inv_std
    kurt = c4 * inv_std2 * inv_std2
    rows = s1_ref.shape[0]
    h = jnp.concatenate(
        [xs_ref[...], mean, std, skew, kurt,
         jnp.broadcast_to(xu_ref[...], (rows, 128))], axis=1)
    z = jnp.dot(_leaky(jnp.dot(h, w3_ref[...], preferred_element_type=jnp.float32)
                       + b3_ref[...]),
                w4_ref[...], preferred_element_type=jnp.float32) + b4_ref[...]
    r = jax.lax.rsqrt(jnp.mean(z * z, axis=-1, keepdims=True) + EPS_F32)
    out_ref[...] = z * r * g_ref[...]


def _node_stage(s1, s2, s3, s4, cnt, x_s, x_u, W3, b3, W4, b4, g, tile=1000):
    n = x_s.shape[0]
    grid = n // tile
    mspec = pl.BlockSpec((tile, 144), lambda i: (i, 0))
    return pl.pallas_call(
        _node_body,
        grid=(grid,),
        in_specs=[
            mspec, mspec, mspec, mspec,
            pl.BlockSpec((tile, 1), lambda i: (i, 0)),
            pl.BlockSpec((tile, 128), lambda i: (i, 0)),
            pl.BlockSpec((1, 128), lambda i: (0, 0)),
            pl.BlockSpec((832, 832), lambda i: (0, 0)),
            pl.BlockSpec((1, 832), lambda i: (0, 0)),
            pl.BlockSpec((832, 128), lambda i: (0, 0)),
            pl.BlockSpec((1, 128), lambda i: (0, 0)),
            pl.BlockSpec((1, 128), lambda i: (0, 0)),
        ],
        out_specs=pl.BlockSpec((tile, 128), lambda i: (i, 0)),
        out_shape=jax.ShapeDtypeStruct((n, 128), jnp.float32),
        interpret=_INTERP,
    )(s1, s2, s3, s4, cnt, x_s, x_u, W3, b3.reshape(1, -1), W4,
      b4.reshape(1, -1), g.reshape(1, -1))


def kernel(x_s, x_t, edge_index, edge_attr, x_u, W1, b1, W2, b2, W3, b3, W4, b4, g):
    n = x_s.shape[0]
    src = edge_index[0].astype(jnp.int32)
    tgt = edge_index[1].astype(jnp.int32)

    xtg = jnp.take(x_t, tgt, axis=0)
    msg = _edge_stage(xtg, edge_attr, W1, b1, W2, b2)

    m2 = msg * msg
    s1 = jax.ops.segment_sum(msg, src, num_segments=n)
    s2 = jax.ops.segment_sum(m2, src, num_segments=n)
    s3 = jax.ops.segment_sum(m2 * msg, src, num_segments=n)
    s4 = jax.ops.segment_sum(m2 * m2, src, num_segments=n)
    cnt = jnp.maximum(
        jax.ops.segment_sum(jnp.ones((src.shape[0],), jnp.float32), src,
                            num_segments=n), 1.0)

    return _node_stage(s1, s2, s3, s4, cnt.reshape(n, 1), x_s, x_u,
                       W3, b3, W4, b4, g)


# speedup vs baseline: 3.3270x; 2.7688x over previous
import functools

import jax
import jax.numpy as jnp
from jax import lax
from jax.experimental import pallas as pl
from jax.experimental.pallas import tpu as pltpu
from jax.experimental.pallas import tpu_sc as plsc

LEAKY_SLOPE = 0.2
EPS_F32 = 1.1920929e-07
_NC, _NS = 2, 16
_NW = _NC * _NS
_N = 10000
_NP = 10240
_E = 320000


def _leaky(x):
    return jnp.where(x >= 0, x, LEAKY_SLOPE * x)


def _sc_gather(x_t, tgt):
    E = tgt.shape[0]
    per_w = E // _NW
    C = 80
    n_chunks = per_w // C
    mesh = plsc.VectorSubcoreMesh(core_axis_name="c", subcore_axis_name="s")

    @functools.partial(
        pl.kernel,
        out_type=jax.ShapeDtypeStruct((E, 128), jnp.float32),
        mesh=mesh,
        scratch_types=[
            pltpu.VMEM((per_w,), jnp.int32),
            pltpu.VMEM((C, 128), jnp.float32),
            pltpu.SemaphoreType.DMA,
        ],
    )
    def k(xt_hbm, tgt_hbm, out_hbm, idx_v, rows_v, sem):
        wid = lax.axis_index("s") * _NC + lax.axis_index("c")
        base = wid * per_w
        pltpu.sync_copy(tgt_hbm.at[pl.ds(base, per_w)], idx_v)

        def body(i, carry):
            pltpu.async_copy(
                xt_hbm.at[idx_v.at[pl.ds(i * C, C)]], rows_v, sem).wait()
            pltpu.sync_copy(rows_v, out_hbm.at[pl.ds(base + i * C, C)])
            return carry

        lax.fori_loop(0, n_chunks, body, 0)

    return k(x_t, tgt)


def _sc_counts(src):
    E = src.shape[0]
    per_w = E // _NW
    C = 80
    n_chunks = per_w // C
    rpt = _NP // _NS
    mesh = plsc.VectorSubcoreMesh(core_axis_name="c", subcore_axis_name="s")

    @functools.partial(
        pl.kernel,
        out_type=jax.ShapeDtypeStruct((2, _NP, 16), jnp.float32),
        mesh=mesh,
        scratch_types=[
            pltpu.VMEM((per_w,), jnp.int32),
            pltpu.VMEM((C, 16), jnp.float32),
            pltpu.VMEM((64, 16), jnp.float32),
            pltpu.VMEM_SHARED((_NP, 16), jnp.float32),
        ],
        compiler_params=pltpu.CompilerParams(use_tc_tiling_on_sc=False),
    )
    def k(src_hbm, cnt_out, idx_v, ones_v, zc_v, cacc):
        c = lax.axis_index("c")
        s = lax.axis_index("s")
        wid = s * _NC + c

        def fill(i, _, ref=None, val=0.0):
            ref[i, :] = jnp.full((16,), val, jnp.float32)
            return _

        lax.fori_loop(0, C, functools.partial(fill, ref=ones_v, val=1.0), 0)
        lax.fori_loop(0, 64, functools.partial(fill, ref=zc_v, val=0.0), 0)
        pltpu.sync_copy(src_hbm.at[pl.ds(wid * per_w, per_w)], idx_v)
        for j in range(10):
            pltpu.sync_copy(zc_v, cacc.at[pl.ds(s * rpt + j * 64, 64)])
        plsc.subcore_barrier()

        def body(i, carry):
            pltpu.sync_copy(ones_v, cacc.at[idx_v.at[pl.ds(i * C, C)]],
                            add=True)
            return carry

        lax.fori_loop(0, n_chunks, body, 0)
        plsc.subcore_barrier()

        @pl.when(c == 0)
        def _():
            pltpu.sync_copy(cacc.at[pl.ds(s * rpt, rpt)],
                            cnt_out.at[0, pl.ds(s * rpt, rpt)])

        @pl.when(c == 1)
        def _():
            pltpu.sync_copy(cacc.at[pl.ds(s * rpt, rpt)],
                            cnt_out.at[1, pl.ds(s * rpt, rpt)])

    return k(src)


def _sc_moments(msg, src):
    E = msg.shape[0]
    C = 80
    B = 4000
    per_t = E // _NS
    n_blocks = per_t // B
    cpb = B // C
    rpt = _NP // _NS
    mesh = plsc.VectorSubcoreMesh(core_axis_name="c", subcore_axis_name="s")

    @functools.partial(
        pl.kernel,
        out_type=(
            jax.ShapeDtypeStruct((2, _NP, 144), jnp.float32),
            jax.ShapeDtypeStruct((2, _NP, 144), jnp.float32),
        ),
        mesh=mesh,
        scratch_types=[
            pltpu.VMEM((B,), jnp.int32),
            pltpu.VMEM((C, 144), jnp.float32),
            pltpu.VMEM((40, 144), jnp.float32),
            pltpu.VMEM_SHARED((_NP, 144), jnp.float32),
        ],
        compiler_params=pltpu.CompilerParams(use_tc_tiling_on_sc=False),
    )
    def k(msg_hbm, src_hbm, out_a, out_b, idx_v, buf, z_v, acc):
        c = lax.axis_index("c")
        s = lax.axis_index("s")

        def zrow(i, _):
            for j in range(9):
                z_v[i, pl.ds(j * 16, 16)] = jnp.zeros((16,), jnp.float32)
            return _

        lax.fori_loop(0, 40, zrow, 0)

        for p in range(2):
            for j in range(16):
                pltpu.sync_copy(z_v, acc.at[pl.ds(s * rpt + j * 40, 40)])
            plsc.subcore_barrier()

            def block(b, carry, p=p):
                base = s * per_t + b * B
                pltpu.sync_copy(src_hbm.at[pl.ds(base, B)], idx_v)

                def chunk(i, carry2):
                    pltpu.sync_copy(msg_hbm.at[pl.ds(base + i * C, C)], buf)
                    if p == 0:
                        @pl.when(c == 1)
                        def _():
                            def prow(r, cy):
                                for j in range(9):
                                    v = buf[r, pl.ds(j * 16, 16)]
                                    buf[r, pl.ds(j * 16, 16)] = v * v
                                return cy
                            lax.fori_loop(0, C, prow, 0)
                    else:
                        @pl.when(c == 0)
                        def _():
                            def prow(r, cy):
                                for j in range(9):
                                    v = buf[r, pl.ds(j * 16, 16)]
                                    buf[r, pl.ds(j * 16, 16)] = v * v * v
                                return cy
                            lax.fori_loop(0, C, prow, 0)

                        @pl.when(c == 1)
                        def _():
                            def prow(r, cy):
                                for j in range(9):
                                    v = buf[r, pl.ds(j * 16, 16)]
                                    v2 = v * v
                                    buf[r, pl.ds(j * 16, 16)] = v2 * v2
                                return cy
                            lax.fori_loop(0, C, prow, 0)

                    pltpu.sync_copy(buf, acc.at[idx_v.at[pl.ds(i * C, C)]],
                                    add=True)
                    return carry2

                lax.fori_loop(0, cpb, chunk, 0)
                return carry

            lax.fori_loop(0, n_blocks, block, 0)
            plsc.subcore_barrier()

            @pl.when(c == 0)
            def _(p=p):
                pltpu.sync_copy(acc.at[pl.ds(s * rpt, rpt)],
                                out_a.at[p, pl.ds(s * rpt, rpt)])

            @pl.when(c == 1)
            def _(p=p):
                pltpu.sync_copy(acc.at[pl.ds(s * rpt, rpt)],
                                out_b.at[p, pl.ds(s * rpt, rpt)])

            plsc.subcore_barrier()

    return k(msg, src)


def _edge_body(xtg_ref, ea_ref, w1_ref, b1_ref, w2_ref, b2_ref, msg_ref):
    a = (
        jnp.dot(xtg_ref[...], w1_ref[0:128, :], preferred_element_type=jnp.float32)
        + jnp.dot(ea_ref[...], w1_ref[128:144, :], preferred_element_type=jnp.float32)
        + b1_ref[...]
    )
    msg_ref[...] = (
        jnp.dot(_leaky(a), w2_ref[...], preferred_element_type=jnp.float32)
        + b2_ref[...]
    )


def _edge_stage(xtg, ea, W1, b1, W2, b2, tile=1280):
    E = xtg.shape[0]
    grid = E // tile
    return pl.pallas_call(
        _edge_body,
        grid=(grid,),
        in_specs=[
            pl.BlockSpec((tile, 128), lambda i: (i, 0)),
            pl.BlockSpec((tile, 16), lambda i: (i, 0)),
            pl.BlockSpec((144, 144), lambda i: (0, 0)),
            pl.BlockSpec((1, 144), lambda i: (0, 0)),
            pl.BlockSpec((144, 144), lambda i: (0, 0)),
            pl.BlockSpec((1, 144), lambda i: (0, 0)),
        ],
        out_specs=pl.BlockSpec((tile, 144), lambda i: (i, 0)),
        out_shape=jax.ShapeDtypeStruct((E, 144), jnp.float32),
    )(xtg, ea, W1, b1.reshape(1, -1), W2, b2.reshape(1, -1))


def _node_body(s1_ref, s2_ref, s3_ref, s4_ref, cnt_ref, xs_ref, xu_ref,
               w3_ref, b3_ref, w4_ref, b4_ref, g_ref, out_ref):
    cnt = jnp.maximum(cnt_ref[0, :, 0:1] + cnt_ref[1, :, 0:1], 1.0)
    inv = 1.0 / cnt
    mean = s1_ref[...] * inv
    m2 = s2_ref[...] * inv
    m3 = s3_ref[...] * inv
    m4 = s4_ref[...] * inv
    mean2 = mean * mean
    var = _leaky(m2 - mean2)
    std = jnp.sqrt(var + 1e-6)
    c3 = m3 - 3.0 * mean * m2 + 2.0 * mean * mean2
    c4 = m4 - 4.0 * mean * m3 + 6.0 * mean2 * m2 - 3.0 * mean2 * mean2
    inv_std = 1.0 / std
    inv_std2 = inv_std * inv_std
    skew = c3 * inv_std2 * inv_std
    kurt = c4 * inv_std2 * inv_std2
    rows = s1_ref.shape[0]
    h = jnp.concatenate(
        [xs_ref[...], mean, std, skew, kurt,
         jnp.broadcast_to(xu_ref[...], (rows, 128))], axis=1)
    z = jnp.dot(_leaky(jnp.dot(h, w3_ref[...], preferred_element_type=jnp.float32)
                       + b3_ref[...]),
                w4_ref[...], preferred_element_type=jnp.float32) + b4_ref[...]
    r = jax.lax.rsqrt(jnp.mean(z * z, axis=-1, keepdims=True) + EPS_F32)
    out_ref[...] = z * r * g_ref[...]


def _node_stage(s1, s2, s3, s4, cnt2, x_s, x_u, W3, b3, W4, b4, g, tile=1000):
    n = x_s.shape[0]
    grid = n // tile
    mspec = pl.BlockSpec((tile, 144), lambda i: (i, 0))
    return pl.pallas_call(
        _node_body,
        grid=(grid,),
        in_specs=[
            mspec, mspec, mspec, mspec,
            pl.BlockSpec((2, tile, 16), lambda i: (0, i, 0)),
            pl.BlockSpec((tile, 128), lambda i: (i, 0)),
            pl.BlockSpec((1, 128), lambda i: (0, 0)),
            pl.BlockSpec((832, 832), lambda i: (0, 0)),
            pl.BlockSpec((1, 832), lambda i: (0, 0)),
            pl.BlockSpec((832, 128), lambda i: (0, 0)),
            pl.BlockSpec((1, 128), lambda i: (0, 0)),
            pl.BlockSpec((1, 128), lambda i: (0, 0)),
        ],
        out_specs=pl.BlockSpec((tile, 128), lambda i: (i, 0)),
        out_shape=jax.ShapeDtypeStruct((n, 128), jnp.float32),
    )(s1, s2, s3, s4, cnt2, x_s, x_u, W3, b3.reshape(1, -1), W4,
      b4.reshape(1, -1), g.reshape(1, -1))


def kernel(x_s, x_t, edge_index, edge_attr, x_u, W1, b1, W2, b2, W3, b3, W4, b4, g):
    src = edge_index[0].astype(jnp.int32)
    tgt = edge_index[1].astype(jnp.int32)

    xtg = _sc_gather(x_t, tgt)
    cnt2 = _sc_counts(src)
    msg = _edge_stage(xtg, edge_attr, W1, b1, W2, b2)
    mom_a, mom_b = _sc_moments(msg, src)

    return _node_stage(mom_a[0, :_N], mom_b[0, :_N], mom_a[1, :_N],
                       mom_b[1, :_N], cnt2[:, :_N],
                       x_s, x_u, W3, b3, W4, b4, g)


# speedup vs baseline: 3.6827x; 1.1069x over previous
import functools

import jax
import jax.numpy as jnp
from jax import lax
from jax.experimental import pallas as pl
from jax.experimental.pallas import tpu as pltpu
from jax.experimental.pallas import tpu_sc as plsc

LEAKY_SLOPE = 0.2
EPS_F32 = 1.1920929e-07
_NC, _NS = 2, 16
_NW = _NC * _NS
_N = 10000
_NP = 10240
_E = 320000
_HW = 80


def _leaky(x):
    return jnp.where(x >= 0, x, LEAKY_SLOPE * x)


def _sc_gather(x_t, tgt):
    E = tgt.shape[0]
    per_w = E // _NW
    C = 80
    n_chunks = per_w // C
    mesh = plsc.VectorSubcoreMesh(core_axis_name="c", subcore_axis_name="s")

    @functools.partial(
        pl.kernel,
        out_type=jax.ShapeDtypeStruct((E, 128), jnp.float32),
        mesh=mesh,
        scratch_types=[
            pltpu.VMEM((per_w,), jnp.int32),
            pltpu.VMEM((C, 128), jnp.float32),
            pltpu.SemaphoreType.DMA,
        ],
    )
    def k(xt_hbm, tgt_hbm, out_hbm, idx_v, rows_v, sem):
        wid = lax.axis_index("s") * _NC + lax.axis_index("c")
        base = wid * per_w
        pltpu.sync_copy(tgt_hbm.at[pl.ds(base, per_w)], idx_v)

        def body(i, carry):
            pltpu.async_copy(
                xt_hbm.at[idx_v.at[pl.ds(i * C, C)]], rows_v, sem).wait()
            pltpu.sync_copy(rows_v, out_hbm.at[pl.ds(base + i * C, C)])
            return carry

        lax.fori_loop(0, n_chunks, body, 0)

    return k(x_t, tgt)


def _sc_counts(src):
    E = src.shape[0]
    per_w = E // _NW
    C = 80
    n_chunks = per_w // C
    rpt = _NP // _NS
    mesh = plsc.VectorSubcoreMesh(core_axis_name="c", subcore_axis_name="s")

    @functools.partial(
        pl.kernel,
        out_type=jax.ShapeDtypeStruct((2, _NP, 16), jnp.float32),
        mesh=mesh,
        scratch_types=[
            pltpu.VMEM((per_w,), jnp.int32),
            pltpu.VMEM((C, 16), jnp.float32),
            pltpu.VMEM((64, 16), jnp.float32),
            pltpu.VMEM_SHARED((_NP, 16), jnp.float32),
        ],
        compiler_params=pltpu.CompilerParams(use_tc_tiling_on_sc=False),
    )
    def k(src_hbm, cnt_out, idx_v, ones_v, zc_v, cacc):
        c = lax.axis_index("c")
        s = lax.axis_index("s")
        wid = s * _NC + c

        def fill(i, _, ref=None, val=0.0):
            ref[i, :] = jnp.full((16,), val, jnp.float32)
            return _

        lax.fori_loop(0, C, functools.partial(fill, ref=ones_v, val=1.0), 0)
        lax.fori_loop(0, 64, functools.partial(fill, ref=zc_v, val=0.0), 0)
        pltpu.sync_copy(src_hbm.at[pl.ds(wid * per_w, per_w)], idx_v)
        for j in range(10):
            pltpu.sync_copy(zc_v, cacc.at[pl.ds(s * rpt + j * 64, 64)])
        plsc.subcore_barrier()

        def body(i, carry):
            pltpu.sync_copy(ones_v, cacc.at[idx_v.at[pl.ds(i * C, C)]],
                            add=True)
            return carry

        lax.fori_loop(0, n_chunks, body, 0)
        plsc.subcore_barrier()

        @pl.when(c == 0)
        def _():
            pltpu.sync_copy(cacc.at[pl.ds(s * rpt, rpt)],
                            cnt_out.at[0, pl.ds(s * rpt, rpt)])

        @pl.when(c == 1)
        def _():
            pltpu.sync_copy(cacc.at[pl.ds(s * rpt, rpt)],
                            cnt_out.at[1, pl.ds(s * rpt, rpt)])

    return k(src)


def _sc_moments(msg, src):
    E = msg.shape[0]
    C = 40
    B = 4000
    per_t = E // _NS
    n_blocks = per_t // B
    cpb = B // C
    rpt = _NP // _NS
    mesh = plsc.VectorSubcoreMesh(core_axis_name="c", subcore_axis_name="s")

    @functools.partial(
        pl.kernel,
        out_type=(
            jax.ShapeDtypeStruct((2, _NP, 160), jnp.float32),
            jax.ShapeDtypeStruct((2, _NP, 160), jnp.float32),
        ),
        mesh=mesh,
        scratch_types=[
            pltpu.VMEM((B,), jnp.int32),
            pltpu.VMEM((2, C, 160), jnp.float32),
            pltpu.SemaphoreType.DMA,
            pltpu.SemaphoreType.DMA,
            pltpu.SemaphoreType.DMA,
            pltpu.SemaphoreType.DMA,
            pltpu.VMEM_SHARED((_NP, 160), jnp.float32),
        ],
        compiler_params=pltpu.CompilerParams(use_tc_tiling_on_sc=False),
    )
    def k(msg_hbm, src_hbm, out_a, out_b,
          idx_v, buf2, l0, l1, s0, s1, acc):
        c = lax.axis_index("c")
        s = lax.axis_index("s")
        colbase = c * _HW
        lsem = (l0, l1)
        ssem = (s0, s1)

        def load_args(base, i, slot):
            return (msg_hbm.at[pl.ds(base + i * C, C),
                               pl.ds(colbase, _HW)],
                    buf2.at[slot, :, pl.ds(0, _HW)],
                    lsem[slot])

        def scat_args(i, slot):
            return (buf2.at[slot],
                    acc.at[idx_v.at[pl.ds(i * C, C)]],
                    ssem[slot])

        def compute(slot, p):
            def prow(r, cy):
                for rr in range(2):
                    for j in range(_HW // 16):
                        sl = pl.ds(j * 16, 16)
                        sh = pl.ds(_HW + j * 16, 16)
                        v = buf2[slot, 2 * r + rr, sl]
                        if p == 0:
                            buf2[slot, 2 * r + rr, sh] = v * v
                        else:
                            v2 = v * v
                            buf2[slot, 2 * r + rr, sl] = v2 * v
                            buf2[slot, 2 * r + rr, sh] = v2 * v2
                return cy

            lax.fori_loop(0, C // 2, prow, 0)

        def zero_acc():
            def zrow(i, carry):
                for j in range(10):
                    buf2[0, i, pl.ds(j * 16, 16)] = jnp.zeros(
                        (16,), jnp.float32)
                return carry

            lax.fori_loop(0, C, zrow, 0)
            for j in range(16):
                pltpu.sync_copy(buf2.at[0], acc.at[pl.ds(s * rpt + j * C, C)])

        for p in range(2):
            zero_acc()
            plsc.subcore_barrier()

            def block(b, carry, p=p):
                base = s * per_t + b * B
                pltpu.sync_copy(src_hbm.at[pl.ds(base, B)], idx_v)
                pltpu.async_copy(*load_args(base, 0, 0))

                def step(i, carry2):
                    def body(slot, other):
                        pltpu.make_async_copy(*load_args(base, i, slot)).wait()

                        @pl.when(i >= 1)
                        def _():
                            pltpu.make_async_copy(
                                *scat_args(i - 1, other)).wait()

                        @pl.when(i <= cpb - 2)
                        def _():
                            pltpu.async_copy(*load_args(base, i + 1, other))

                        compute(slot, p)
                        pltpu.async_copy(*scat_args(i, slot), add=True)

                    @pl.when(i % 2 == 0)
                    def _():
                        body(0, 1)

                    @pl.when(i % 2 == 1)
                    def _():
                        body(1, 0)

                    return carry2

                lax.fori_loop(0, cpb, step, 0)
                pltpu.make_async_copy(*scat_args(cpb - 1, (cpb - 1) % 2)).wait()
                return carry

            lax.fori_loop(0, n_blocks, block, 0)
            plsc.subcore_barrier()

            @pl.when(c == 0)
            def _(p=p):
                pltpu.sync_copy(acc.at[pl.ds(s * rpt, rpt)],
                                out_a.at[p, pl.ds(s * rpt, rpt)])

            @pl.when(c == 1)
            def _(p=p):
                pltpu.sync_copy(acc.at[pl.ds(s * rpt, rpt)],
                                out_b.at[p, pl.ds(s * rpt, rpt)])

            plsc.subcore_barrier()

    return k(msg, src)


def _edge_body(xtg_ref, ea_ref, w1_ref, b1_ref, w2_ref, b2_ref, msg_ref):
    a = (
        jnp.dot(xtg_ref[...], w1_ref[0:128, :], preferred_element_type=jnp.float32)
        + jnp.dot(ea_ref[...], w1_ref[128:144, :], preferred_element_type=jnp.float32)
        + b1_ref[...]
    )
    m = (
        jnp.dot(_leaky(a), w2_ref[...], preferred_element_type=jnp.float32)
        + b2_ref[...]
    )
    rows = m.shape[0]
    msg_ref[...] = jnp.concatenate(
        [m, jnp.zeros((rows, 16), jnp.float32)], axis=1)


def _edge_stage(xtg, ea, W1, b1, W2, b2, tile=1280):
    E = xtg.shape[0]
    grid = E // tile
    return pl.pallas_call(
        _edge_body,
        grid=(grid,),
        in_specs=[
            pl.BlockSpec((tile, 128), lambda i: (i, 0)),
            pl.BlockSpec((tile, 16), lambda i: (i, 0)),
            pl.BlockSpec((144, 144), lambda i: (0, 0)),
            pl.BlockSpec((1, 144), lambda i: (0, 0)),
            pl.BlockSpec((144, 144), lambda i: (0, 0)),
            pl.BlockSpec((1, 144), lambda i: (0, 0)),
        ],
        out_specs=pl.BlockSpec((tile, 160), lambda i: (i, 0)),
        out_shape=jax.ShapeDtypeStruct((E, 160), jnp.float32),
    )(xtg, ea, W1, b1.reshape(1, -1), W2, b2.reshape(1, -1))


def _node_body(ma_ref, mb_ref, cnt_ref, xs_ref, xu_ref,
               w3_ref, b3_ref, w4_ref, b4_ref, g_ref, out_ref):
    s1 = jnp.concatenate([ma_ref[0, :, 0:80], mb_ref[0, :, 0:64]], axis=1)
    s2 = jnp.concatenate([ma_ref[0, :, 80:160], mb_ref[0, :, 80:144]], axis=1)
    s3 = jnp.concatenate([ma_ref[1, :, 0:80], mb_ref[1, :, 0:64]], axis=1)
    s4 = jnp.concatenate([ma_ref[1, :, 80:160], mb_ref[1, :, 80:144]], axis=1)
    cnt = jnp.maximum(cnt_ref[0, :, 0:1] + cnt_ref[1, :, 0:1], 1.0)
    inv = 1.0 / cnt
    mean = s1 * inv
    m2 = s2 * inv
    m3 = s3 * inv
    m4 = s4 * inv
    mean2 = mean * mean
    var = _leaky(m2 - mean2)
    std = jnp.sqrt(var + 1e-6)
    c3 = m3 - 3.0 * mean * m2 + 2.0 * mean * mean2
    c4 = m4 - 4.0 * mean * m3 + 6.0 * mean2 * m2 - 3.0 * mean2 * mean2
    inv_std = 1.0 / std
    inv_std2 = inv_std * inv_std
    skew = c3 * inv_std2 * inv_std
    kurt = c4 * inv_std2 * inv_std2
    rows = xs_ref.shape[0]
    h = jnp.concatenate(
        [xs_ref[...], mean, std, skew, kurt,
         jnp.broadcast_to(xu_ref[...], (rows, 128))], axis=1)
    z = jnp.dot(_leaky(jnp.dot(h, w3_ref[...], preferred_element_type=jnp.float32)
                       + b3_ref[...]),
                w4_ref[...], preferred_element_type=jnp.float32) + b4_ref[...]
    r = jax.lax.rsqrt(jnp.mean(z * z, axis=-1, keepdims=True) + EPS_F32)
    out_ref[...] = z * r * g_ref[...]


def _node_stage(mom_a, mom_b, cnt2, x_s, x_u, W3, b3, W4, b4, g, tile=1000):
    n = x_s.shape[0]
    grid = n // tile
    mspec = pl.BlockSpec((2, tile, 160), lambda i: (0, i, 0))
    return pl.pallas_call(
        _node_body,
        grid=(grid,),
        in_specs=[
            mspec, mspec,
            pl.BlockSpec((2, tile, 16), lambda i: (0, i, 0)),
            pl.BlockSpec((tile, 128), lambda i: (i, 0)),
            pl.BlockSpec((1, 128), lambda i: (0, 0)),
            pl.BlockSpec((832, 832), lambda i: (0, 0)),
            pl.BlockSpec((1, 832), lambda i: (0, 0)),
            pl.BlockSpec((832, 128), lambda i: (0, 0)),
            pl.BlockSpec((1, 128), lambda i: (0, 0)),
            pl.BlockSpec((1, 128), lambda i: (0, 0)),
        ],
        out_specs=pl.BlockSpec((tile, 128), lambda i: (i, 0)),
        out_shape=jax.ShapeDtypeStruct((n, 128), jnp.float32),
    )(mom_a, mom_b, cnt2, x_s, x_u, W3, b3.reshape(1, -1), W4,
      b4.reshape(1, -1), g.reshape(1, -1))


def kernel(x_s, x_t, edge_index, edge_attr, x_u, W1, b1, W2, b2, W3, b3, W4, b4, g):
    src = edge_index[0].astype(jnp.int32)
    tgt = edge_index[1].astype(jnp.int32)

    xtg = _sc_gather(x_t, tgt)
    cnt2 = _sc_counts(src)
    msg = _edge_stage(xtg, edge_attr, W1, b1, W2, b2)
    mom_a, mom_b = _sc_moments(msg, src)

    return _node_stage(mom_a[:, :_N], mom_b[:, :_N], cnt2[:, :_N],
                       x_s, x_u, W3, b3, W4, b4, g)


# speedup vs baseline: 4.8113x; 1.3065x over previous
import functools

import jax
import jax.numpy as jnp
from jax import lax
from jax.experimental import pallas as pl
from jax.experimental.pallas import tpu as pltpu
from jax.experimental.pallas import tpu_sc as plsc

LEAKY_SLOPE = 0.2
EPS_F32 = 1.1920929e-07
_NC, _NS = 2, 16
_NW = _NC * _NS
_N = 10000
_NP = 10240
_E = 320000
_HW = 80


def _leaky(x):
    return jnp.where(x >= 0, x, LEAKY_SLOPE * x)


def _sc_gather(x_t, tgt, goff, half):
    per_w = half // _NW
    C = 40
    n_chunks = per_w // C
    mesh = plsc.VectorSubcoreMesh(core_axis_name="c", subcore_axis_name="s")

    @functools.partial(
        pl.kernel,
        out_type=jax.ShapeDtypeStruct((half, 128), jnp.float32),
        mesh=mesh,
        scratch_types=[
            pltpu.VMEM((per_w,), jnp.int32),
            pltpu.VMEM((C, 128), jnp.float32),
            pltpu.SemaphoreType.DMA,
        ],
    )
    def k(xt_hbm, tgt_hbm, out_hbm, idx_v, rows_v, sem):
        wid = lax.axis_index("s") * _NC + lax.axis_index("c")
        base = wid * per_w
        pltpu.sync_copy(tgt_hbm.at[pl.ds(goff + base, per_w)], idx_v)

        def body(i, carry):
            pltpu.async_copy(
                xt_hbm.at[idx_v.at[pl.ds(i * C, C)]], rows_v, sem).wait()
            pltpu.sync_copy(rows_v, out_hbm.at[pl.ds(base + i * C, C)])
            return carry

        lax.fori_loop(0, n_chunks, body, 0)

    return k(x_t, tgt)


def _sc_counts(src):
    E = src.shape[0]
    per_w = E // _NW
    C = 80
    n_chunks = per_w // C
    rpt = _NP // _NS
    mesh = plsc.VectorSubcoreMesh(core_axis_name="c", subcore_axis_name="s")

    @functools.partial(
        pl.kernel,
        out_type=jax.ShapeDtypeStruct((2, _NP, 16), jnp.float32),
        mesh=mesh,
        scratch_types=[
            pltpu.VMEM((per_w,), jnp.int32),
            pltpu.VMEM((C, 16), jnp.float32),
            pltpu.VMEM((64, 16), jnp.float32),
            pltpu.VMEM_SHARED((_NP, 16), jnp.float32),
        ],
        compiler_params=pltpu.CompilerParams(use_tc_tiling_on_sc=False),
    )
    def k(src_hbm, cnt_out, idx_v, ones_v, zc_v, cacc):
        c = lax.axis_index("c")
        s = lax.axis_index("s")
        wid = s * _NC + c

        def fill(i, _, ref=None, val=0.0):
            ref[i, :] = jnp.full((16,), val, jnp.float32)
            return _

        lax.fori_loop(0, C, functools.partial(fill, ref=ones_v, val=1.0), 0)
        lax.fori_loop(0, 64, functools.partial(fill, ref=zc_v, val=0.0), 0)
        pltpu.sync_copy(src_hbm.at[pl.ds(wid * per_w, per_w)], idx_v)
        for j in range(10):
            pltpu.sync_copy(zc_v, cacc.at[pl.ds(s * rpt + j * 64, 64)])
        plsc.subcore_barrier()

        def body(i, carry):
            pltpu.sync_copy(ones_v, cacc.at[idx_v.at[pl.ds(i * C, C)]],
                            add=True)
            return carry

        lax.fori_loop(0, n_chunks, body, 0)
        plsc.subcore_barrier()

        @pl.when(c == 0)
        def _():
            pltpu.sync_copy(cacc.at[pl.ds(s * rpt, rpt)],
                            cnt_out.at[0, pl.ds(s * rpt, rpt)])

        @pl.when(c == 1)
        def _():
            pltpu.sync_copy(cacc.at[pl.ds(s * rpt, rpt)],
                            cnt_out.at[1, pl.ds(s * rpt, rpt)])

    return k(src)


def _sc_moments(msg, src):
    E = msg.shape[0]
    C = 80
    B = 2000
    per_t = E // _NS
    n_blocks = per_t // B
    cpb = B // C
    rpt = _NP // _NS
    mesh = plsc.VectorSubcoreMesh(core_axis_name="c", subcore_axis_name="s")

    @functools.partial(
        pl.kernel,
        out_type=(
            jax.ShapeDtypeStruct((2, _NP, 160), jnp.float32),
            jax.ShapeDtypeStruct((2, _NP, 160), jnp.float32),
        ),
        mesh=mesh,
        scratch_types=[
            pltpu.VMEM((B,), jnp.int32),
            pltpu.VMEM((2, C, 160), jnp.float32),
            pltpu.SemaphoreType.DMA,
            pltpu.SemaphoreType.DMA,
            pltpu.SemaphoreType.DMA,
            pltpu.SemaphoreType.DMA,
            pltpu.VMEM_SHARED((_NP, 160), jnp.float32),
        ],
        compiler_params=pltpu.CompilerParams(use_tc_tiling_on_sc=False),
    )
    def k(msg_hbm, src_hbm, out_a, out_b,
          idx_v, buf2, l0, l1, s0, s1, acc):
        c = lax.axis_index("c")
        s = lax.axis_index("s")
        colbase = c * _HW
        lsem = (l0, l1)
        ssem = (s0, s1)

        def load_args(base, i, slot):
            return (msg_hbm.at[pl.ds(base + i * C, C),
                               pl.ds(colbase, _HW)],
                    buf2.at[slot, :, pl.ds(0, _HW)],
                    lsem[slot])

        def scat_args(i, slot):
            return (buf2.at[slot],
                    acc.at[idx_v.at[pl.ds(i * C, C)]],
                    ssem[slot])

        def compute(slot, p):
            def prow(r, cy):
                for rr in range(2):
                    for j in range(_HW // 16):
                        sl = pl.ds(j * 16, 16)
                        sh = pl.ds(_HW + j * 16, 16)
                        v = buf2[slot, 2 * r + rr, sl]
                        if p == 0:
                            buf2[slot, 2 * r + rr, sh] = v * v
                        else:
                            v2 = v * v
                            buf2[slot, 2 * r + rr, sl] = v2 * v
                            buf2[slot, 2 * r + rr, sh] = v2 * v2
                return cy

            lax.fori_loop(0, C // 2, prow, 0)

        def zero_acc():
            def zrow(i, carry):
                for j in range(10):
                    buf2[0, i, pl.ds(j * 16, 16)] = jnp.zeros(
                        (16,), jnp.float32)
                return carry

            lax.fori_loop(0, C, zrow, 0)
            for j in range(rpt // C):
                pltpu.sync_copy(buf2.at[0], acc.at[pl.ds(s * rpt + j * C, C)])

        for p in range(2):
            zero_acc()
            plsc.subcore_barrier()

            def block(b, carry, p=p):
                base = s * per_t + b * B
                pltpu.sync_copy(src_hbm.at[pl.ds(base, B)], idx_v)
                pltpu.async_copy(*load_args(base, 0, 0))

                def step(i, carry2):
                    def body(slot):
                        nxt = (slot + 1) % 2
                        pltpu.make_async_copy(*load_args(base, i, slot)).wait()

                        @pl.when(i >= 1)
                        def _():
                            pltpu.make_async_copy(
                                *scat_args(i - 1, nxt)).wait()

                        @pl.when(i <= cpb - 2)
                        def _():
                            pltpu.async_copy(*load_args(base, i + 1, nxt))

                        compute(slot, p)
                        pltpu.async_copy(*scat_args(i, slot), add=True)

                    for kk in range(2):
                        @pl.when(i % 2 == kk)
                        def _(kk=kk):
                            body(kk)

                    return carry2

                lax.fori_loop(0, cpb, step, 0)
                pltpu.make_async_copy(*scat_args(cpb - 1, (cpb - 1) % 2)).wait()
                return carry

            lax.fori_loop(0, n_blocks, block, 0)
            plsc.subcore_barrier()

            @pl.when(c == 0)
            def _(p=p):
                pltpu.sync_copy(acc.at[pl.ds(s * rpt, rpt)],
                                out_a.at[p, pl.ds(s * rpt, rpt)])

            @pl.when(c == 1)
            def _(p=p):
                pltpu.sync_copy(acc.at[pl.ds(s * rpt, rpt)],
                                out_b.at[p, pl.ds(s * rpt, rpt)])

            plsc.subcore_barrier()

    return k(msg, src)


def _edge_body(xtg_ref, ea_ref, w1_ref, b1_ref, w2_ref, b2_ref, msg_ref):
    a = (
        jnp.dot(xtg_ref[...], w1_ref[0:128, :], preferred_element_type=jnp.float32)
        + jnp.dot(ea_ref[...], w1_ref[128:144, :], preferred_element_type=jnp.float32)
        + b1_ref[...]
    )
    m = (
        jnp.dot(_leaky(a), w2_ref[...], preferred_element_type=jnp.float32)
        + b2_ref[...]
    )
    rows = m.shape[0]
    msg_ref[...] = jnp.concatenate(
        [m, jnp.zeros((rows, 16), jnp.float32)], axis=1)


def _edge_stage(xtg, ea, W1, b1, W2, b2, row_off, tile=1280):
    E = xtg.shape[0]
    grid = E // tile
    ob = row_off // tile
    return pl.pallas_call(
        _edge_body,
        grid=(grid,),
        in_specs=[
            pl.BlockSpec((tile, 128), lambda i: (i, 0)),
            pl.BlockSpec((tile, 16), lambda i, ob=ob: (i + ob, 0)),
            pl.BlockSpec((144, 144), lambda i: (0, 0)),
            pl.BlockSpec((1, 144), lambda i: (0, 0)),
            pl.BlockSpec((144, 144), lambda i: (0, 0)),
            pl.BlockSpec((1, 144), lambda i: (0, 0)),
        ],
        out_specs=pl.BlockSpec((tile, 160), lambda i: (i, 0)),
        out_shape=jax.ShapeDtypeStruct((E, 160), jnp.float32),
    )(xtg, ea, W1, b1.reshape(1, -1), W2, b2.reshape(1, -1))


def _node_body(ma0_ref, ma1_ref, mb0_ref, mb1_ref, cnt_ref, xs_ref, xu_ref,
               w3_ref, b3_ref, w4_ref, b4_ref, g_ref, out_ref):
    ma = ma0_ref[...] + ma1_ref[...]
    mb = mb0_ref[...] + mb1_ref[...]
    s1 = jnp.concatenate([ma[0, :, 0:80], mb[0, :, 0:64]], axis=1)
    s2 = jnp.concatenate([ma[0, :, 80:160], mb[0, :, 80:144]], axis=1)
    s3 = jnp.concatenate([ma[1, :, 0:80], mb[1, :, 0:64]], axis=1)
    s4 = jnp.concatenate([ma[1, :, 80:160], mb[1, :, 80:144]], axis=1)
    cnt = jnp.maximum(cnt_ref[0, :, 0:1] + cnt_ref[1, :, 0:1], 1.0)
    inv = 1.0 / cnt
    mean = s1 * inv
    m2 = s2 * inv
    m3 = s3 * inv
    m4 = s4 * inv
    mean2 = mean * mean
    var = _leaky(m2 - mean2)
    std = jnp.sqrt(var + 1e-6)
    c3 = m3 - 3.0 * mean * m2 + 2.0 * mean * mean2
    c4 = m4 - 4.0 * mean * m3 + 6.0 * mean2 * m2 - 3.0 * mean2 * mean2
    inv_std = 1.0 / std
    inv_std2 = inv_std * inv_std
    skew = c3 * inv_std2 * inv_std
    kurt = c4 * inv_std2 * inv_std2
    rows = xs_ref.shape[0]
    h = jnp.concatenate(
        [xs_ref[...], mean, std, skew, kurt,
         jnp.broadcast_to(xu_ref[...], (rows, 128))], axis=1)
    z = jnp.dot(_leaky(jnp.dot(h, w3_ref[...], preferred_element_type=jnp.float32)
                       + b3_ref[...]),
                w4_ref[...], preferred_element_type=jnp.float32) + b4_ref[...]
    r = jax.lax.rsqrt(jnp.mean(z * z, axis=-1, keepdims=True) + EPS_F32)
    out_ref[...] = z * r * g_ref[...]


def _node_stage(ma0, ma1, mb0, mb1, cnt2, x_s, x_u, W3, b3, W4, b4, g,
                tile=1000):
    n = x_s.shape[0]
    grid = n // tile
    mspec = pl.BlockSpec((2, tile, 160), lambda i: (0, i, 0))
    return pl.pallas_call(
        _node_body,
        grid=(grid,),
        in_specs=[
            mspec, mspec, mspec, mspec,
            pl.BlockSpec((2, tile, 16), lambda i: (0, i, 0)),
            pl.BlockSpec((tile, 128), lambda i: (i, 0)),
            pl.BlockSpec((1, 128), lambda i: (0, 0)),
            pl.BlockSpec((832, 832), lambda i: (0, 0)),
            pl.BlockSpec((1, 832), lambda i: (0, 0)),
            pl.BlockSpec((832, 128), lambda i: (0, 0)),
            pl.BlockSpec((1, 128), lambda i: (0, 0)),
            pl.BlockSpec((1, 128), lambda i: (0, 0)),
        ],
        out_specs=pl.BlockSpec((tile, 128), lambda i: (i, 0)),
        out_shape=jax.ShapeDtypeStruct((n, 128), jnp.float32),
    )(ma0, ma1, mb0, mb1, cnt2, x_s, x_u, W3, b3.reshape(1, -1), W4,
      b4.reshape(1, -1), g.reshape(1, -1))


def kernel(x_s, x_t, edge_index, edge_attr, x_u, W1, b1, W2, b2, W3, b3, W4, b4, g):
    src = edge_index[0].astype(jnp.int32)
    tgt = edge_index[1].astype(jnp.int32)
    half = _E // 2

    cnt2 = _sc_counts(src)
    xtg0 = _sc_gather(x_t, tgt, 0, half)
    xtg1 = _sc_gather(x_t, tgt, half, half)
    msg0 = _edge_stage(xtg0, edge_attr, W1, b1, W2, b2, 0)
    msg1 = _edge_stage(xtg1, edge_attr, W1, b1, W2, b2, half)
    ma0, mb0 = _sc_moments(msg0, src[:half])
    ma1, mb1 = _sc_moments(msg1, src[half:])

    return _node_stage(ma0, ma1, mb0, mb1, cnt2,
                       x_s, x_u, W3, b3, W4, b4, g)
